# initial kernel scaffold (unmeasured)
import jax
import jax.numpy as jnp
from jax import lax
from jax.experimental import pallas as pl
from jax.experimental.pallas import tpu as pltpu

N_DEV = 4
SQ = 2048
SKV_LOC = 2048
D = 1024
H = 8
DH = 128
QT = 512
N_QT = SQ // QT
KC = 512
N_KC = SKV_LOC // KC
BLK = 64
SCALE = 0.08838834764831843


def _body(x_ref, wq_ref, k_ref, v_ref, wo_ref, out_ref,
          q_ref, ocomm_ref, lcomm_ref, otot_ref, ltot_ref,
          osend, orecv, lsend, lrecv):
    my = lax.axis_index("i")
    left = lax.rem(my + N_DEV - 1, N_DEV)
    right = lax.rem(my + 1, N_DEV)
    off = my * SKV_LOC

    q_ref[...] = jnp.dot(x_ref[...], wq_ref[...],
                         preferred_element_type=jnp.float32)

    ocomm_ref[0] = jnp.zeros_like(ocomm_ref[0])
    lcomm_ref[0] = jnp.zeros_like(lcomm_ref[0])

    for h in range(H):
        hs = slice(h * DH, (h + 1) * DH)
        for qt in range(N_QT):
            rows = pl.ds(qt * QT, QT)
            q_tile = q_ref[rows, hs]
            nkc = jnp.clip(((qt + 1) * QT - off) // KC, 0, N_KC)

            def kc_body(kc, _):
                kcols = pl.ds(kc * KC, KC)
                k_c = k_ref[kcols, hs]
                v_c = v_ref[kcols, hs]
                s = lax.dot_general(
                    q_tile, k_c, (((1,), (1,)), ((), ())),
                    preferred_element_type=jnp.float32) * SCALE
                i_idx = lax.broadcasted_iota(jnp.int32, (QT, KC), 0) + qt * QT
                j_idx = (lax.broadcasted_iota(jnp.int32, (QT, KC), 1)
                         + off + kc * KC)
                mask = (j_idx // BLK) <= (i_idx // BLK)
                w = jnp.where(mask, jnp.exp(s), 0.0)
                ocomm_ref[0, rows, hs] = ocomm_ref[0, rows, hs] + lax.dot_general(
                    w, v_c, (((1,), (0,)), ((), ())),
                    preferred_element_type=jnp.float32)
                lcomm_ref[0, rows, h:h + 1] = (
                    lcomm_ref[0, rows, h:h + 1]
                    + jnp.sum(w, axis=1, keepdims=True))
                return 0

            lax.fori_loop(0, nkc, kc_body, 0)

    otot_ref[...] = ocomm_ref[0]
    ltot_ref[...] = lcomm_ref[0]

    barrier = pltpu.get_barrier_semaphore()
    for nbr in (left, right):
        pl.semaphore_signal(barrier, inc=1, device_id=(nbr,),
                            device_id_type=pl.DeviceIdType.MESH)
    pl.semaphore_wait(barrier, 2)

    for hop in range(N_DEV - 1):
        s_slot = hop % 2
        r_slot = (hop + 1) % 2
        o_rdma = pltpu.make_async_remote_copy(
            src_ref=ocomm_ref.at[s_slot], dst_ref=ocomm_ref.at[r_slot],
            send_sem=osend.at[s_slot], recv_sem=orecv.at[r_slot],
            device_id=(right,), device_id_type=pl.DeviceIdType.MESH)
        l_rdma = pltpu.make_async_remote_copy(
            src_ref=lcomm_ref.at[s_slot], dst_ref=lcomm_ref.at[r_slot],
            send_sem=lsend.at[s_slot], recv_sem=lrecv.at[r_slot],
            device_id=(right,), device_id_type=pl.DeviceIdType.MESH)
        o_rdma.start()
        l_rdma.start()
        o_rdma.wait()
        l_rdma.wait()
        otot_ref[...] = otot_ref[...] + ocomm_ref[r_slot]
        ltot_ref[...] = ltot_ref[...] + lcomm_ref[r_slot]

    for h in range(H):
        hs = slice(h * DH, (h + 1) * DH)
        ocomm_ref[0, :, hs] = otot_ref[:, hs] / ltot_ref[:, h:h + 1]
    out_ref[...] = jnp.dot(ocomm_ref[0], wo_ref[...],
                           preferred_element_type=jnp.float32)


def kernel(x, Wq, K_ext, V_ext, Wo):
    x2 = x.reshape(SQ, D)
    k2 = K_ext.reshape(SKV_LOC, H * DH)
    v2 = V_ext.reshape(SKV_LOC, H * DH)
    out = pl.pallas_call(
        _body,
        out_shape=jax.ShapeDtypeStruct((SQ, D), jnp.float32),
        in_specs=[pl.BlockSpec(memory_space=pltpu.VMEM)] * 5,
        out_specs=pl.BlockSpec(memory_space=pltpu.VMEM),
        scratch_shapes=[
            pltpu.VMEM((SQ, D), jnp.float32),
            pltpu.VMEM((2, SQ, D), jnp.float32),
            pltpu.VMEM((2, SQ, 128), jnp.float32),
            pltpu.VMEM((SQ, D), jnp.float32),
            pltpu.VMEM((SQ, 128), jnp.float32),
            pltpu.SemaphoreType.DMA((2,)),
            pltpu.SemaphoreType.DMA((2,)),
            pltpu.SemaphoreType.DMA((2,)),
            pltpu.SemaphoreType.DMA((2,)),
        ],
        compiler_params=pltpu.CompilerParams(collective_id=0),
    )(x2, Wq, k2, v2, Wo)
    return out.reshape(1, SQ, D)


# baseline (device time: 411988 ns/iter reference)
import jax
import jax.numpy as jnp
from jax import lax
from jax.experimental import pallas as pl
from jax.experimental.pallas import tpu as pltpu

N_DEV = 4
SQ = 2048
SKV_LOC = 2048
D = 1024
H = 8
DH = 128
QT = 512
N_QT = SQ // QT
KC = 512
N_KC = SKV_LOC // KC
BLK = 64
SCALE = 0.08838834764831843


def _body(x_hbm_ref, wq_ref, k_ref, v_ref, wo_ref, out_ref,
          xtile_ref, qtile_ref, ocomm_ref, lcomm_ref, ltot_ref,
          xsem, osend, orecv, lsend, lrecv):
    my = lax.axis_index("i")
    left = lax.rem(my + N_DEV - 1, N_DEV)
    right = lax.rem(my + 1, N_DEV)
    off = my * SKV_LOC

    ocomm_ref[0] = jnp.zeros_like(ocomm_ref[0])
    lcomm_ref[0] = jnp.zeros_like(lcomm_ref[0])

    for qt in range(N_QT):
        rows = pl.ds(qt * QT, QT)
        cp = pltpu.make_async_copy(x_hbm_ref.at[rows], xtile_ref, xsem)
        cp.start()
        cp.wait()
        qtile_ref[...] = jnp.dot(xtile_ref[...], wq_ref[...],
                                 preferred_element_type=jnp.float32)
        nkc = jnp.clip(((qt + 1) * QT - off) // KC, 0, N_KC)
        for h in range(H):
            hs = slice(h * DH, (h + 1) * DH)
            q_tile = qtile_ref[:, hs]

            def kc_body(kc, _, q_tile=q_tile, hs=hs, rows=rows, qt=qt, h=h):
                kcols = pl.ds(kc * KC, KC)
                k_c = k_ref[kcols, hs]
                v_c = v_ref[kcols, hs]
                s = lax.dot_general(
                    q_tile, k_c, (((1,), (1,)), ((), ())),
                    preferred_element_type=jnp.float32) * SCALE
                i_idx = lax.broadcasted_iota(jnp.int32, (QT, KC), 0) + qt * QT
                j_idx = (lax.broadcasted_iota(jnp.int32, (QT, KC), 1)
                         + off + kc * KC)
                mask = (j_idx // BLK) <= (i_idx // BLK)
                w = jnp.where(mask, jnp.exp(s), 0.0)
                ocomm_ref[0, rows, hs] = ocomm_ref[0, rows, hs] + lax.dot_general(
                    w, v_c, (((1,), (0,)), ((), ())),
                    preferred_element_type=jnp.float32)
                lcomm_ref[0, rows, h:h + 1] = (
                    lcomm_ref[0, rows, h:h + 1]
                    + jnp.sum(w, axis=1, keepdims=True))
                return 0

            lax.fori_loop(0, nkc, kc_body, 0)

    out_ref[...] = ocomm_ref[0]
    ltot_ref[...] = lcomm_ref[0]

    barrier = pltpu.get_barrier_semaphore()
    for nbr in (left, right):
        pl.semaphore_signal(barrier, inc=1, device_id=(nbr,),
                            device_id_type=pl.DeviceIdType.MESH)
    pl.semaphore_wait(barrier, 2)

    for hop in range(N_DEV - 1):
        s_slot = hop % 2
        r_slot = (hop + 1) % 2
        o_rdma = pltpu.make_async_remote_copy(
            src_ref=ocomm_ref.at[s_slot], dst_ref=ocomm_ref.at[r_slot],
            send_sem=osend.at[s_slot], recv_sem=orecv.at[r_slot],
            device_id=(right,), device_id_type=pl.DeviceIdType.MESH)
        l_rdma = pltpu.make_async_remote_copy(
            src_ref=lcomm_ref.at[s_slot], dst_ref=lcomm_ref.at[r_slot],
            send_sem=lsend.at[s_slot], recv_sem=lrecv.at[r_slot],
            device_id=(right,), device_id_type=pl.DeviceIdType.MESH)
        o_rdma.start()
        l_rdma.start()
        o_rdma.wait()
        l_rdma.wait()
        out_ref[...] = out_ref[...] + ocomm_ref[r_slot]
        ltot_ref[...] = ltot_ref[...] + lcomm_ref[r_slot]

    for h in range(H):
        hs = slice(h * DH, (h + 1) * DH)
        ocomm_ref[0, :, hs] = out_ref[:, hs] / ltot_ref[:, h:h + 1]
    out_ref[...] = jnp.dot(ocomm_ref[0], wo_ref[...],
                           preferred_element_type=jnp.float32)


def kernel(x, Wq, K_ext, V_ext, Wo):
    x2 = x.reshape(SQ, D)
    k2 = K_ext.reshape(SKV_LOC, H * DH)
    v2 = V_ext.reshape(SKV_LOC, H * DH)
    out = pl.pallas_call(
        _body,
        out_shape=jax.ShapeDtypeStruct((SQ, D), jnp.float32),
        in_specs=[
            pl.BlockSpec(memory_space=pl.ANY),
            pl.BlockSpec(memory_space=pltpu.VMEM),
            pl.BlockSpec(memory_space=pltpu.VMEM),
            pl.BlockSpec(memory_space=pltpu.VMEM),
            pl.BlockSpec(memory_space=pltpu.VMEM),
        ],
        out_specs=pl.BlockSpec(memory_space=pltpu.VMEM),
        scratch_shapes=[
            pltpu.VMEM((QT, D), jnp.float32),
            pltpu.VMEM((QT, D), jnp.float32),
            pltpu.VMEM((2, SQ, D), jnp.float32),
            pltpu.VMEM((2, SQ, 128), jnp.float32),
            pltpu.VMEM((SQ, 128), jnp.float32),
            pltpu.SemaphoreType.DMA,
            pltpu.SemaphoreType.DMA((2,)),
            pltpu.SemaphoreType.DMA((2,)),
            pltpu.SemaphoreType.DMA((2,)),
            pltpu.SemaphoreType.DMA((2,)),
        ],
        compiler_params=pltpu.CompilerParams(
            collective_id=0,
            vmem_limit_bytes=100 * 1024 * 1024,
        ),
    )(x2, Wq, k2, v2, Wo)
    return out.reshape(1, SQ, D)


# device time: 158742 ns/iter; 2.5953x vs baseline; 2.5953x over previous
import functools

import jax
import jax.numpy as jnp
from jax import lax
from jax.experimental import pallas as pl
from jax.experimental.pallas import tpu as pltpu

N_DEV = 4
SQ = 2048
SKV_LOC = 2048
D = 1024
H = 8
DH = 128
QT = 512
N_QT = SQ // QT
KC = 512
BLK = 64
SCALE = 0.08838834764831843

_MESH = pl.DeviceIdType.MESH


def _body(x_hbm_ref, wq_ref, k_ref, v_ref, wo_ref, out_ref,
          xtile_ref, qtile_ref, oacc_ref, lacc_ref,
          xsem, ssemA, ssemB, rsem):
    my = lax.axis_index("i")
    left = lax.rem(my + N_DEV - 1, N_DEV)
    right = lax.rem(my + 1, N_DEV)

    barrier = pltpu.get_barrier_semaphore()
    for nbr in (left, right):
        pl.semaphore_signal(barrier, inc=1, device_id=(nbr,),
                            device_id_type=_MESH)
    pl.semaphore_wait(barrier, 2)

    def tile_rdma(qt, send_sems, target):
        rows = pl.ds(qt * QT, QT)
        return pltpu.make_async_remote_copy(
            src_ref=out_ref.at[rows], dst_ref=out_ref.at[rows],
            send_sem=send_sems.at[qt], recv_sem=rsem.at[qt],
            device_id=(target,), device_id_type=_MESH)

    @pl.when(my == 0)
    def _producer():
        sends = []
        for qt in range(N_QT):
            rows = pl.ds(qt * QT, QT)
            cp = pltpu.make_async_copy(x_hbm_ref.at[rows], xtile_ref, xsem)
            cp.start()
            cp.wait()
            qtile_ref[...] = jnp.dot(xtile_ref[...], wq_ref[...],
                                     preferred_element_type=jnp.float32)
            oacc_ref[...] = jnp.zeros_like(oacc_ref)
            lacc_ref[...] = jnp.zeros_like(lacc_ref)
            for kc in range(qt + 1):
                kr = pl.ds(kc * KC, KC)
                for h in range(H):
                    hs = slice(h * DH, (h + 1) * DH)
                    s = lax.dot_general(
                        qtile_ref[:, hs], k_ref[kr, hs],
                        (((1,), (1,)), ((), ())),
                        preferred_element_type=jnp.float32) * SCALE
                    if kc == qt:
                        i_blk = lax.broadcasted_iota(jnp.int32, (QT, KC), 0) // BLK
                        j_blk = lax.broadcasted_iota(jnp.int32, (QT, KC), 1) // BLK
                        w = jnp.where(j_blk <= i_blk, jnp.exp(s), 0.0)
                    else:
                        w = jnp.exp(s)
                    oacc_ref[:, hs] = oacc_ref[:, hs] + lax.dot_general(
                        w, v_ref[kr, hs], (((1,), (0,)), ((), ())),
                        preferred_element_type=jnp.float32)
                    lacc_ref[:, h:h + 1] = (lacc_ref[:, h:h + 1]
                                            + jnp.sum(w, axis=1, keepdims=True))
            for h in range(H):
                hs = slice(h * DH, (h + 1) * DH)
                xtile_ref[:, hs] = oacc_ref[:, hs] / lacc_ref[:, h:h + 1]
            out_ref[rows, :] = jnp.dot(xtile_ref[...], wo_ref[...],
                                       preferred_element_type=jnp.float32)
            for send_sems, target in ((ssemA, 1), (ssemB, 3)):
                rdma = tile_rdma(qt, send_sems, target)
                rdma.start()
                sends.append(rdma)
        for rdma in sends:
            rdma.wait_send()

    @pl.when(my == 1)
    def _forwarder():
        fwds = []
        for qt in range(N_QT):
            tile_rdma(qt, ssemA, 0).wait_recv()
            fwd = tile_rdma(qt, ssemB, 2)
            fwd.start()
            fwds.append(fwd)
        for fwd in fwds:
            fwd.wait_send()

    @pl.when(my >= 2)
    def _receivers():
        for qt in range(N_QT):
            tile_rdma(qt, ssemA, 0).wait_recv()

    @functools.partial(pl.run_scoped, exit_sem=pltpu.SemaphoreType.REGULAR)
    def _(exit_sem):
        for nbr in (left, right):
            pl.semaphore_signal(exit_sem, inc=1, device_id=(nbr,),
                                device_id_type=_MESH)
        pl.semaphore_wait(exit_sem, 2)


def kernel(x, Wq, K_ext, V_ext, Wo):
    x2 = x.reshape(SQ, D)
    k2 = K_ext.reshape(SKV_LOC, H * DH)
    v2 = V_ext.reshape(SKV_LOC, H * DH)
    out = pl.pallas_call(
        _body,
        out_shape=jax.ShapeDtypeStruct((SQ, D), jnp.float32),
        in_specs=[
            pl.BlockSpec(memory_space=pl.ANY),
            pl.BlockSpec(memory_space=pltpu.VMEM),
            pl.BlockSpec(memory_space=pltpu.VMEM),
            pl.BlockSpec(memory_space=pltpu.VMEM),
            pl.BlockSpec(memory_space=pltpu.VMEM),
        ],
        out_specs=pl.BlockSpec(memory_space=pltpu.VMEM),
        scratch_shapes=[
            pltpu.VMEM((QT, D), jnp.float32),
            pltpu.VMEM((QT, D), jnp.float32),
            pltpu.VMEM((QT, D), jnp.float32),
            pltpu.VMEM((QT, 128), jnp.float32),
            pltpu.SemaphoreType.DMA,
            pltpu.SemaphoreType.DMA((N_QT,)),
            pltpu.SemaphoreType.DMA((N_QT,)),
            pltpu.SemaphoreType.DMA((N_QT,)),
        ],
        compiler_params=pltpu.CompilerParams(
            collective_id=0,
            vmem_limit_bytes=100 * 1024 * 1024,
        ),
    )(x2, Wq, k2, v2, Wo)
    return out.reshape(1, SQ, D)


# device time: 148493 ns/iter; 2.7745x vs baseline; 1.0690x over previous
import functools

import jax
import jax.numpy as jnp
from jax import lax
from jax.experimental import pallas as pl
from jax.experimental.pallas import tpu as pltpu

N_DEV = 4
SQ = 2048
SKV_LOC = 2048
D = 1024
H = 8
DH = 128
QT = 512
N_QT = SQ // QT
KC = 512
BLK = 64
HT = QT // 2
SCALE = 0.08838834764831843
N_SEM = N_QT + 1

_MESH = pl.DeviceIdType.MESH


def _body(x_ref, wq_ref, k_ref, v_ref, wo_ref, out_ref,
          ctx_ref, qtile_ref, oacc_ref, lacc_ref,
          ssemA, ssemB, rsem):
    my = lax.axis_index("i")
    left = lax.rem(my + N_DEV - 1, N_DEV)
    right = lax.rem(my + 1, N_DEV)

    barrier = pltpu.get_barrier_semaphore()
    for nbr in (left, right):
        pl.semaphore_signal(barrier, inc=1, device_id=(nbr,),
                            device_id_type=_MESH)
    pl.semaphore_wait(barrier, 2)

    def rows_for(slot):
        if slot < N_QT - 1:
            return pl.ds(slot * QT, QT)
        if slot == N_QT - 1:
            return pl.ds((N_QT - 1) * QT, HT)
        return pl.ds((N_QT - 1) * QT + HT, HT)

    def seg_rdma(slot, send_sems, recv_slot, target):
        return pltpu.make_async_remote_copy(
            src_ref=out_ref.at[rows_for(slot)],
            dst_ref=out_ref.at[rows_for(recv_slot)],
            send_sem=send_sems.at[slot], recv_sem=rsem.at[recv_slot],
            device_id=(target,), device_id_type=_MESH)

    @pl.when(my == 0)
    def _producer():
        sends = []
        for qt in range(N_QT):
            rows = pl.ds(qt * QT, QT)
            qtile_ref[...] = jnp.dot(x_ref[rows, :], wq_ref[...],
                                     preferred_element_type=jnp.float32)
            oacc_ref[...] = jnp.zeros_like(oacc_ref)
            lacc_ref[...] = jnp.zeros_like(lacc_ref)
            for kc in range(qt + 1):
                kr = pl.ds(kc * KC, KC)
                for h in range(H):
                    hs = slice(h * DH, (h + 1) * DH)
                    s = lax.dot_general(
                        qtile_ref[:, hs], k_ref[kr, hs],
                        (((1,), (1,)), ((), ())),
                        preferred_element_type=jnp.float32) * SCALE
                    if kc == qt:
                        i_blk = lax.broadcasted_iota(jnp.int32, (QT, KC), 0) // BLK
                        j_blk = lax.broadcasted_iota(jnp.int32, (QT, KC), 1) // BLK
                        w = jnp.where(j_blk <= i_blk, jnp.exp(s), 0.0)
                    else:
                        w = jnp.exp(s)
                    oacc_ref[:, hs] = oacc_ref[:, hs] + lax.dot_general(
                        w, v_ref[kr, hs], (((1,), (0,)), ((), ())),
                        preferred_element_type=jnp.float32)
                    lacc_ref[:, h:h + 1] = (lacc_ref[:, h:h + 1]
                                            + jnp.sum(w, axis=1, keepdims=True))
            for h in range(H):
                hs = slice(h * DH, (h + 1) * DH)
                ctx_ref[:, hs] = oacc_ref[:, hs] / lacc_ref[:, h:h + 1]
            out_ref[rows, :] = jnp.dot(ctx_ref[...], wo_ref[...],
                                       preferred_element_type=jnp.float32)
            if qt < N_QT - 1:
                plan = (((qt,), ssemA, 1), ((qt,), ssemB, 3))
            else:
                plan = (((3, 4), ssemA, 1), ((4, 3), ssemB, 3))
            for slots, send_sems, target in plan:
                for slot in slots:
                    rdma = seg_rdma(slot, send_sems, slot, target)
                    rdma.start()
                    sends.append(rdma)
        for rdma in sends:
            rdma.wait_send()

    @pl.when(my == 1)
    def _forwarder():
        fwds = []
        for slot in range(N_QT):
            seg_rdma(slot, ssemA, slot, 0).wait_recv()
            fwd = seg_rdma(slot, ssemB, slot, 2)
            fwd.start()
            fwds.append(fwd)
        seg_rdma(4, ssemA, 4, 0).wait_recv()
        for fwd in fwds:
            fwd.wait_send()

    @pl.when(my == 3)
    def _forwarder_b():
        seg_rdma(4, ssemA, 4, 0).wait_recv()
        fwd = seg_rdma(4, ssemB, 4, 2)
        fwd.start()
        for slot in range(N_QT):
            seg_rdma(slot, ssemA, slot, 0).wait_recv()
        fwd.wait_send()

    @pl.when(my == 2)
    def _receiver():
        for slot in range(N_SEM):
            seg_rdma(slot, ssemA, slot, 0).wait_recv()

    @functools.partial(pl.run_scoped, exit_sem=pltpu.SemaphoreType.REGULAR)
    def _(exit_sem):
        for nbr in (left, right):
            pl.semaphore_signal(exit_sem, inc=1, device_id=(nbr,),
                                device_id_type=_MESH)
        pl.semaphore_wait(exit_sem, 2)


def kernel(x, Wq, K_ext, V_ext, Wo):
    x2 = x.reshape(SQ, D)
    k2 = K_ext.reshape(SKV_LOC, H * DH)
    v2 = V_ext.reshape(SKV_LOC, H * DH)
    out = pl.pallas_call(
        _body,
        out_shape=jax.ShapeDtypeStruct((SQ, D), jnp.float32),
        in_specs=[pl.BlockSpec(memory_space=pltpu.VMEM)] * 5,
        out_specs=pl.BlockSpec(memory_space=pltpu.VMEM),
        scratch_shapes=[
            pltpu.VMEM((QT, D), jnp.float32),
            pltpu.VMEM((QT, D), jnp.float32),
            pltpu.VMEM((QT, D), jnp.float32),
            pltpu.VMEM((QT, 128), jnp.float32),
            pltpu.SemaphoreType.DMA((N_SEM,)),
            pltpu.SemaphoreType.DMA((N_SEM,)),
            pltpu.SemaphoreType.DMA((N_SEM,)),
        ],
        compiler_params=pltpu.CompilerParams(
            collective_id=0,
            vmem_limit_bytes=100 * 1024 * 1024,
        ),
    )(x2, Wq, k2, v2, Wo)
    return out.reshape(1, SQ, D)
